# baseline (speedup 1.0000x reference)
import functools
import jax
import jax.numpy as jnp
from jax import lax
from jax.experimental import pallas as pl
from jax.experimental.pallas import tpu as pltpu
from jax.experimental.pallas import tpu_sc as plsc

VOCAB = 100000
EMBED_DIM = 128
BATCH = 4096
SEQ = 50
TOT = BATCH * SEQ

_NC, _NS = 2, 16
NW = _NC * _NS
NPART = 4
PART_B = BATCH // NPART
SENT_W = PART_B // NW
SENT_C = 8
CHUNK = SENT_C * SEQ
NSTEP = SENT_W // SENT_C


def _make_part(part):
    @functools.partial(
        pl.kernel,
        mesh=plsc.VectorSubcoreMesh(core_axis_name="c", subcore_axis_name="s"),
        out_type=jax.ShapeDtypeStruct((PART_B, SEQ, EMBED_DIM), jnp.float32),
        scratch_types=[
            pltpu.VMEM((CHUNK,), jnp.int32),
            pltpu.VMEM((CHUNK,), jnp.int32),
            pltpu.VMEM((CHUNK, EMBED_DIM), jnp.float32),
            pltpu.VMEM((CHUNK, EMBED_DIM), jnp.float32),
            pltpu.SemaphoreType.DMA,
            pltpu.SemaphoreType.DMA,
            pltpu.SemaphoreType.DMA,
            pltpu.SemaphoreType.DMA,
            pltpu.SemaphoreType.DMA,
            pltpu.SemaphoreType.DMA,
        ],
        name=f"embed_gather_p{part}",
    )
    def _gather_kernel(idx_hbm, table_hbm, out_hbm,
                       idx0, idx1, rows0, rows1,
                       si0, si1, sg0, sg1, so0, so1):
        wid = lax.axis_index("s") * _NC + lax.axis_index("c")
        sent_local = wid * SENT_W
        row_base = (part * PART_B + sent_local) * SEQ
        idxv, rows = [idx0, idx1], [rows0, rows1]
        si, sg, so = [si0, si1], [sg0, sg1], [so0, so1]

        def idx_cp(i):
            b = i % 2
            return pltpu.make_async_copy(
                idx_hbm.at[pl.ds(row_base + i * CHUNK, CHUNK)], idxv[b], si[b])

        def gather_cp(i):
            b = i % 2
            return pltpu.make_async_copy(table_hbm.at[idxv[b]], rows[b], sg[b])

        def out_cps(i):
            b = i % 2
            s0 = sent_local + i * SENT_C
            return [
                pltpu.make_async_copy(
                    rows[b].at[pl.ds(s * SEQ, SEQ)], out_hbm.at[s0 + s], so[b])
                for s in range(SENT_C)
            ]

        idx_cp(0).start()
        idx_cp(1).start()
        idx_cp(0).wait()
        gather_cp(0).start()
        for i in range(NSTEP):
            gather_cp(i).wait()
            if i + 2 < NSTEP:
                idx_cp(i + 2).start()
            if i + 1 < NSTEP:
                if i >= 1:
                    for c in out_cps(i - 1):
                        c.wait()
                idx_cp(i + 1).wait()
                gather_cp(i + 1).start()
            for c in out_cps(i):
                c.start()
        for i in (NSTEP - 2, NSTEP - 1):
            for c in out_cps(i):
                c.wait()

    return _gather_kernel


_PART_KERNELS = [_make_part(p) for p in range(NPART)]


def kernel(sentences, table):
    idx = sentences.reshape(TOT).astype(jnp.int32)
    parts = [k(idx, table) for k in _PART_KERNELS]
    return jnp.concatenate(parts, axis=0)


# speedup vs baseline: 3.2030x; 3.2030x over previous
import functools
import jax
import jax.numpy as jnp
from jax import lax
from jax.experimental import pallas as pl
from jax.experimental.pallas import tpu as pltpu
from jax.experimental.pallas import tpu_sc as plsc

VOCAB = 100000
EMBED_DIM = 128
BATCH = 4096
SEQ = 50
TOT = BATCH * SEQ

_NC, _NS = 2, 16
NW = _NC * _NS
PER_W = TOT // NW
CHUNK = 400
NSTEP = PER_W // CHUNK


@functools.partial(
    pl.kernel,
    mesh=plsc.VectorSubcoreMesh(core_axis_name="c", subcore_axis_name="s"),
    out_type=jax.ShapeDtypeStruct((TOT, EMBED_DIM), jnp.float32),
    scratch_types=[
        pltpu.VMEM((CHUNK,), jnp.int32),
        pltpu.VMEM((CHUNK,), jnp.int32),
        pltpu.VMEM((CHUNK, EMBED_DIM), jnp.float32),
        pltpu.VMEM((CHUNK, EMBED_DIM), jnp.float32),
        pltpu.SemaphoreType.DMA,
        pltpu.SemaphoreType.DMA,
        pltpu.SemaphoreType.DMA,
        pltpu.SemaphoreType.DMA,
        pltpu.SemaphoreType.DMA,
        pltpu.SemaphoreType.DMA,
    ],
)
def _gather_kernel(idx_hbm, table_hbm, out_hbm,
                   idx0, idx1, rows0, rows1, si0, si1, sg0, sg1, so0, so1):
    wid = lax.axis_index("s") * _NC + lax.axis_index("c")
    base = wid * PER_W
    idxv, rows = [idx0, idx1], [rows0, rows1]
    si, sg, so = [si0, si1], [sg0, sg1], [so0, so1]

    def idx_cp(i):
        b = i % 2
        return pltpu.make_async_copy(
            idx_hbm.at[pl.ds(base + i * CHUNK, CHUNK)], idxv[b], si[b])

    def gather_cp(i):
        b = i % 2
        return pltpu.make_async_copy(table_hbm.at[idxv[b]], rows[b], sg[b])

    def out_cp(i):
        b = i % 2
        return pltpu.make_async_copy(
            rows[b], out_hbm.at[pl.ds(base + i * CHUNK, CHUNK)], so[b])

    idx_cp(0).start()
    idx_cp(1).start()
    idx_cp(0).wait()
    gather_cp(0).start()
    for i in range(NSTEP):
        gather_cp(i).wait()
        if i + 2 < NSTEP:
            idx_cp(i + 2).start()
        if i + 1 < NSTEP:
            if i >= 1:
                out_cp(i - 1).wait()
            idx_cp(i + 1).wait()
            gather_cp(i + 1).start()
        out_cp(i).start()
    out_cp(NSTEP - 2).wait()
    out_cp(NSTEP - 1).wait()


def kernel(sentences, table):
    idx = sentences.T.reshape(TOT).astype(jnp.int32)
    out = _gather_kernel(idx, table)
    return out.reshape(SEQ, BATCH, EMBED_DIM).transpose(1, 0, 2)
